# baseline (speedup 1.0000x reference)
import functools

import jax
import jax.numpy as jnp
from jax import lax
from jax.experimental import pallas as pl
from jax.experimental.pallas import tpu as pltpu
from jax.experimental.pallas import tpu_sc as plsc

B = 4096
S = 200
D = 64
NTOK = B * S
NC = 2
NS = 16
NW = NC * NS
TOK_PER_W = NTOK // NW
G = 100
NSUB = 4
CH = G * NSUB
NCHUNK = TOK_PER_W // CH


def _sc_body(resp_hbm, els_hbm, lag_hbm, pos_hbm, rtbl_hbm, etbl_hbm,
             ltbl_hbm, sidx_hbm, out_hbm, sidx_v, ridx, eidx, lidx, out_buf):
    wid = lax.axis_index("s") * NC + lax.axis_index("c")
    base_tok = wid * TOK_PER_W
    base_row = wid * (TOK_PER_W // G)

    pltpu.sync_copy(sidx_hbm, sidx_v)

    @pl.loop(0, NCHUNK)
    def _chunk(ci):
        tok0 = base_tok + ci * CH
        r0 = base_row + ci * NSUB
        pltpu.sync_copy(resp_hbm.at[pl.ds(r0, NSUB)], ridx)
        pltpu.sync_copy(els_hbm.at[pl.ds(r0, NSUB)], eidx)
        pltpu.sync_copy(lag_hbm.at[pl.ds(r0, NSUB)], lidx)
        for j in range(NSUB):
            dst = out_buf.at[pl.ds(j * G, G)]
            pltpu.sync_copy(rtbl_hbm.at[ridx.at[j]], dst)
            pltpu.sync_copy(etbl_hbm.at[eidx.at[j]], dst, add=True)
            pltpu.sync_copy(ltbl_hbm.at[lidx.at[j]], dst, add=True)
            pltpu.sync_copy(pos_hbm.at[sidx_v.at[j]], dst, add=True)
        pltpu.sync_copy(out_buf, out_hbm.at[pl.ds(tok0, CH)])


@jax.jit
def _run(resp2d, els2d, lag2d, position_embed, response_embed,
         elapsed_time_embed, lag_time_embed, sidx):
    mesh = plsc.VectorSubcoreMesh(core_axis_name="c", subcore_axis_name="s")
    f = pl.kernel(
        _sc_body,
        out_type=jax.ShapeDtypeStruct((NTOK, D), jnp.float32),
        mesh=mesh,
        compiler_params=pltpu.CompilerParams(use_tc_tiling_on_sc=False),
        scratch_types=[
            pltpu.VMEM((NSUB, G), jnp.int32),
            pltpu.VMEM((NSUB, G), jnp.int32),
            pltpu.VMEM((NSUB, G), jnp.int32),
            pltpu.VMEM((NSUB, G), jnp.int32),
            pltpu.VMEM((CH, D), jnp.float32),
        ],
    )
    return f(resp2d, els2d, lag2d, position_embed, response_embed,
             elapsed_time_embed, lag_time_embed, sidx)


def kernel(response, elapsed_time, lag_time, position_embed, response_embed,
           elapsed_time_embed, lag_time_embed):
    resp2d = response.astype(jnp.int32).reshape(NTOK // G, G)
    els2d = elapsed_time.astype(jnp.int32).reshape(NTOK // G, G)
    lag2d = lag_time.astype(jnp.int32).reshape(NTOK // G, G)
    sidx = (jnp.arange(CH, dtype=jnp.int32) % S).reshape(NSUB, G)
    out = _run(resp2d, els2d, lag2d, position_embed, response_embed,
               elapsed_time_embed, lag_time_embed, sidx)
    return out.reshape(B, S, D)
